# baseline (speedup 1.0000x reference)
import functools

import jax
import jax.numpy as jnp
from jax import lax
from jax.experimental import pallas as pl
from jax.experimental.pallas import tpu as pltpu
from jax.experimental.pallas import tpu_sc as plsc

N = 10000
E = 320000
D = 128
H = 32

CHUNK = 128
NCHUNK = E // CHUNK
NW = 32
HW = H // 2


def _encode_body(x_ref, we_ref, be_ref, wd_ref, bd_ref, yp_ref, l1_ref):
    x = x_ref[...]
    y = jnp.dot(x, we_ref[...], preferred_element_type=jnp.float32)
    y = y + be_ref[...]
    xh = jnp.dot(y, wd_ref[...], preferred_element_type=jnp.float32)
    xh = xh + bd_ref[...]
    d = x - xh
    l1_ref[0, 0] = jnp.sum(d * d)
    r = lax.broadcasted_iota(jnp.int32, (H, HW), 0)
    c = lax.broadcasted_iota(jnp.int32, (H, HW), 1)
    sel_e = (r == 2 * c).astype(jnp.float32)
    sel_o = (r == 2 * c + 1).astype(jnp.float32)
    ye = jnp.dot(y, sel_e, preferred_element_type=jnp.float32)
    yo = jnp.dot(y, sel_o, preferred_element_type=jnp.float32)
    ue = lax.bitcast_convert_type(ye.astype(jnp.bfloat16), jnp.uint16)
    uo = lax.bitcast_convert_type(yo.astype(jnp.bfloat16), jnp.uint16)
    yp_ref[...] = lax.shift_left(uo.astype(jnp.int32), 16) | ue.astype(jnp.int32)


def _encode(x, W_e, b_e, W_d, b_d):
    return pl.pallas_call(
        _encode_body,
        out_shape=(
            jax.ShapeDtypeStruct((N, HW), jnp.int32),
            jax.ShapeDtypeStruct((1, 1), jnp.float32),
        ),
        in_specs=[
            pl.BlockSpec(memory_space=pltpu.VMEM),
            pl.BlockSpec(memory_space=pltpu.VMEM),
            pl.BlockSpec(memory_space=pltpu.VMEM),
            pl.BlockSpec(memory_space=pltpu.VMEM),
            pl.BlockSpec(memory_space=pltpu.VMEM),
        ],
        out_specs=(
            pl.BlockSpec(memory_space=pltpu.VMEM),
            pl.BlockSpec(memory_space=pltpu.SMEM),
        ),
    )(x, W_e, b_e.reshape(1, H), W_d, b_d.reshape(1, D))


EPW = E // NW
SUPER = 512
NSUP = -(-EPW // SUPER)


def _sc_body(y_hbm, row_hbm, col_hbm, ew_hbm, out_hbm,
             idxr_v, idxc_v, w_v, acc_v,
             ra0, rb0, ra1, rb1, ra2, rb2, ra3, rb3,
             sem0, sem1, sem2, sem3):
    cid = lax.axis_index("c")
    sid = lax.axis_index("s")
    wid = sid * 2 + cid
    w0 = wid * EPW
    iota16 = lax.iota(jnp.int32, 16)
    hi_mask = jnp.full((16,), -65536, jnp.int32)

    c1 = pltpu.async_copy(row_hbm.at[pl.ds(w0, EPW)], idxr_v, sem0)
    c2 = pltpu.async_copy(col_hbm.at[pl.ds(w0, EPW)], idxc_v, sem0)
    c3 = pltpu.async_copy(ew_hbm.at[pl.ds(w0, EPW)], w_v, sem0)
    c1.wait()
    c2.wait()
    c3.wait()

    bufs = ((ra0, rb0, sem0), (ra1, rb1, sem1),
            (ra2, rb2, sem2), (ra3, rb3, sem3))
    NBUF = len(bufs)

    def loc_of(t):
        return lax.min(t * SUPER, EPW - SUPER)

    def issue(t, b):
        r1, r2, sem = bufs[b]
        loc = loc_of(t)
        pltpu.async_copy(y_hbm.at[idxr_v.at[pl.ds(loc, SUPER)]], r1, sem)
        pltpu.async_copy(y_hbm.at[idxc_v.at[pl.ds(loc, SUPER)]], r2, sem)

    def drain(b):
        r1, r2, sem = bufs[b]
        pltpu.make_async_copy(y_hbm.at[idxr_v.at[pl.ds(0, SUPER)]], r1, sem).wait()
        pltpu.make_async_copy(y_hbm.at[idxc_v.at[pl.ds(0, SUPER)]], r2, sem).wait()

    def compute(t, b, acc2):
        r1, r2, _ = bufs[b]
        loc = loc_of(t)

        def group(g, a):
            e_idx = iota16 + g * 16
            accs = [jnp.zeros((32,), jnp.bfloat16) for _ in range(4)]
            for d2 in range(HW):
                d_vec = jnp.full((16,), d2, jnp.int32)
                v1 = plsc.load_gather(r1, [e_idx, d_vec])
                v2 = plsc.load_gather(r2, [e_idx, d_vec])
                d = plsc.bitcast(v1, jnp.bfloat16) - plsc.bitcast(v2, jnp.bfloat16)
                k = d2 % 4
                accs[k] = accs[k] + d * d
            acc32 = (accs[0] + accs[1]) + (accs[2] + accs[3])
            ai = plsc.bitcast(acc32, jnp.int32)
            lo = plsc.bitcast(lax.shift_left(ai, 16), jnp.float32)
            hi = plsc.bitcast(ai, jnp.float32)
            acc = lo + hi
            tq = acc + 1e-12
            i = plsc.bitcast(tq, jnp.int32)
            y = plsc.bitcast(jnp.int32(0x5F3759DF) - (i >> 1), jnp.float32)
            htq = tq * -0.5
            for _ in range(3):
                y = y * (htq * (y * y) + 1.5)
            dist = tq * y
            wv = w_v[pl.ds(loc + g * 16, 16)]
            fresh = loc + g * 16 >= t * SUPER
            contrib = jnp.where(jnp.full((16,), fresh), dist * wv, 0.0)
            return a + contrib

        return lax.fori_loop(0, SUPER // 16, group, acc2)

    for b in range(NBUF):
        issue(b, b)

    def step(p, acc2):
        t0 = p * NBUF
        for u in range(NBUF):
            t = t0 + u
            drain(u)
            acc2 = compute(t, u, acc2)

            @pl.when(t + NBUF < NSUP)
            def _():
                issue(t + NBUF, u)

        return acc2

    acc2 = lax.fori_loop(0, NSUP // NBUF, step, jnp.zeros((16,), jnp.float32))
    acc_v[...] = acc2
    pltpu.sync_copy(acc_v, out_hbm.at[wid])


_sc_edge_loss = functools.partial(
    pl.kernel,
    out_type=jax.ShapeDtypeStruct((NW, 16), jnp.float32),
    mesh=plsc.VectorSubcoreMesh(core_axis_name="c", subcore_axis_name="s"),
    scratch_types=[
        pltpu.VMEM((EPW,), jnp.int32),
        pltpu.VMEM((EPW,), jnp.int32),
        pltpu.VMEM((EPW,), jnp.float32),
        pltpu.VMEM((16,), jnp.float32),
        pltpu.VMEM((SUPER, HW), jnp.int32),
        pltpu.VMEM((SUPER, HW), jnp.int32),
        pltpu.VMEM((SUPER, HW), jnp.int32),
        pltpu.VMEM((SUPER, HW), jnp.int32),
        pltpu.VMEM((SUPER, HW), jnp.int32),
        pltpu.VMEM((SUPER, HW), jnp.int32),
        pltpu.VMEM((SUPER, HW), jnp.int32),
        pltpu.VMEM((SUPER, HW), jnp.int32),
        pltpu.SemaphoreType.DMA,
        pltpu.SemaphoreType.DMA,
        pltpu.SemaphoreType.DMA,
        pltpu.SemaphoreType.DMA,
    ],
    compiler_params=pltpu.CompilerParams(
        needs_layout_passes=False, use_tc_tiling_on_sc=False),
)(_sc_body)


def _finalize_body(part_ref, l1_ref, out_ref):
    s2 = jnp.sum(part_ref[...])
    out_ref[0] = l1_ref[0, 0] * (1.0 / (N * D)) + s2 * (1.0 / E)


def _finalize(partials, l1):
    return pl.pallas_call(
        _finalize_body,
        out_shape=jax.ShapeDtypeStruct((1,), jnp.float32),
        in_specs=[
            pl.BlockSpec(memory_space=pltpu.VMEM),
            pl.BlockSpec(memory_space=pltpu.SMEM),
        ],
        out_specs=pl.BlockSpec(memory_space=pltpu.SMEM),
    )(partials, l1)


def kernel(x, edge_index, edge_weight, W_e, b_e, W_d, b_d):
    y_packed, l1 = _encode(x, W_e, b_e, W_d, b_d)
    row = edge_index[0]
    col = edge_index[1]
    partials = _sc_edge_loss(y_packed, row, col, edge_weight)
    out = _finalize(partials, l1)
    return out[0]


# speedup vs baseline: 1.0309x; 1.0309x over previous
import functools

import jax
import jax.numpy as jnp
from jax import lax
from jax.experimental import pallas as pl
from jax.experimental.pallas import tpu as pltpu
from jax.experimental.pallas import tpu_sc as plsc

N = 10000
E = 320000
D = 128
H = 32

CHUNK = 128
NCHUNK = E // CHUNK
NW = 32
HW = H // 2


def _encode_body(x_ref, we_ref, be_ref, wd_ref, bd_ref, yp_ref, l1_ref):
    x = x_ref[...]
    y = jnp.dot(x, we_ref[...], preferred_element_type=jnp.float32)
    y = y + be_ref[...]
    xh = jnp.dot(y, wd_ref[...], preferred_element_type=jnp.float32)
    xh = xh + bd_ref[...]
    d = x - xh
    l1_ref[0, 0] = jnp.sum(d * d)
    r = lax.broadcasted_iota(jnp.int32, (H, HW), 0)
    c = lax.broadcasted_iota(jnp.int32, (H, HW), 1)
    sel_e = (r == 2 * c).astype(jnp.float32)
    sel_o = (r == 2 * c + 1).astype(jnp.float32)
    ye = jnp.dot(y, sel_e, preferred_element_type=jnp.float32)
    yo = jnp.dot(y, sel_o, preferred_element_type=jnp.float32)
    ue = lax.bitcast_convert_type(ye.astype(jnp.bfloat16), jnp.uint16)
    uo = lax.bitcast_convert_type(yo.astype(jnp.bfloat16), jnp.uint16)
    yp_ref[...] = lax.shift_left(uo.astype(jnp.int32), 16) | ue.astype(jnp.int32)


def _encode(x, W_e, b_e, W_d, b_d):
    return pl.pallas_call(
        _encode_body,
        out_shape=(
            jax.ShapeDtypeStruct((N, HW), jnp.int32),
            jax.ShapeDtypeStruct((1, 1), jnp.float32),
        ),
        in_specs=[
            pl.BlockSpec(memory_space=pltpu.VMEM),
            pl.BlockSpec(memory_space=pltpu.VMEM),
            pl.BlockSpec(memory_space=pltpu.VMEM),
            pl.BlockSpec(memory_space=pltpu.VMEM),
            pl.BlockSpec(memory_space=pltpu.VMEM),
        ],
        out_specs=(
            pl.BlockSpec(memory_space=pltpu.VMEM),
            pl.BlockSpec(memory_space=pltpu.SMEM),
        ),
    )(x, W_e, b_e.reshape(1, H), W_d, b_d.reshape(1, D))


EPW = E // NW
SUPER = 512
NSUP = -(-EPW // SUPER)


def _sc_body(y_hbm, row_hbm, col_hbm, ew_hbm, out_hbm,
             idxr_v, idxc_v, w_v, acc_v, ra0, rb0, ra1, rb1, sem0, sem1):
    cid = lax.axis_index("c")
    sid = lax.axis_index("s")
    wid = sid * 2 + cid
    w0 = wid * EPW
    iota16 = lax.iota(jnp.int32, 16)
    hi_mask = jnp.full((16,), -65536, jnp.int32)

    c1 = pltpu.async_copy(row_hbm.at[pl.ds(w0, EPW)], idxr_v, sem0)
    c2 = pltpu.async_copy(col_hbm.at[pl.ds(w0, EPW)], idxc_v, sem0)
    c3 = pltpu.async_copy(ew_hbm.at[pl.ds(w0, EPW)], w_v, sem0)
    c1.wait()
    c2.wait()
    c3.wait()

    bufs = ((ra0, rb0, sem0), (ra1, rb1, sem1))

    def loc_of(t):
        return lax.min(t * SUPER, EPW - SUPER)

    def issue(t, b):
        r1, r2, sem = bufs[b]
        loc = loc_of(t)
        pltpu.async_copy(y_hbm.at[idxr_v.at[pl.ds(loc, SUPER)]], r1, sem)
        pltpu.async_copy(y_hbm.at[idxc_v.at[pl.ds(loc, SUPER)]], r2, sem)

    def drain(b):
        r1, r2, sem = bufs[b]
        pltpu.make_async_copy(y_hbm.at[idxr_v.at[pl.ds(0, SUPER)]], r1, sem).wait()
        pltpu.make_async_copy(y_hbm.at[idxc_v.at[pl.ds(0, SUPER)]], r2, sem).wait()

    def compute(t, b, acc2):
        r1, r2, _ = bufs[b]
        loc = loc_of(t)

        def group(g, a):
            e_idx = iota16 + g * 16
            accs = [jnp.zeros((32,), jnp.bfloat16) for _ in range(4)]
            for d2 in range(HW):
                d_vec = jnp.full((16,), d2, jnp.int32)
                v1 = plsc.load_gather(r1, [e_idx, d_vec])
                v2 = plsc.load_gather(r2, [e_idx, d_vec])
                d = plsc.bitcast(v1, jnp.bfloat16) - plsc.bitcast(v2, jnp.bfloat16)
                k = d2 % 4
                accs[k] = accs[k] + d * d
            acc32 = (accs[0] + accs[1]) + (accs[2] + accs[3])
            ai = plsc.bitcast(acc32, jnp.int32)
            lo = plsc.bitcast(lax.shift_left(ai, 16), jnp.float32)
            hi = plsc.bitcast(ai, jnp.float32)
            acc = lo + hi
            tq = acc + 1e-12
            i = plsc.bitcast(tq, jnp.int32)
            y = plsc.bitcast(jnp.int32(0x5F3759DF) - (i >> 1), jnp.float32)
            htq = tq * -0.5
            for _ in range(3):
                y = y * (htq * (y * y) + 1.5)
            dist = tq * y
            wv = w_v[pl.ds(loc + g * 16, 16)]
            fresh = loc + g * 16 >= t * SUPER
            contrib = jnp.where(jnp.full((16,), fresh), dist * wv, 0.0)
            return a + contrib

        return lax.fori_loop(0, SUPER // 16, group, acc2)

    issue(0, 0)

    def step(p, acc2):
        t0 = p * 2
        issue(t0 + 1, 1)
        drain(0)
        acc2 = compute(t0, 0, acc2)

        @pl.when(t0 + 2 < NSUP)
        def _():
            issue(t0 + 2, 0)

        drain(1)
        acc2 = compute(t0 + 1, 1, acc2)
        return acc2

    acc2 = lax.fori_loop(0, NSUP // 2, step, jnp.zeros((16,), jnp.float32))
    acc_v[...] = acc2
    pltpu.sync_copy(acc_v, out_hbm.at[wid])


_sc_edge_loss = functools.partial(
    pl.kernel,
    out_type=jax.ShapeDtypeStruct((NW, 16), jnp.float32),
    mesh=plsc.VectorSubcoreMesh(core_axis_name="c", subcore_axis_name="s"),
    scratch_types=[
        pltpu.VMEM((EPW,), jnp.int32),
        pltpu.VMEM((EPW,), jnp.int32),
        pltpu.VMEM((EPW,), jnp.float32),
        pltpu.VMEM((16,), jnp.float32),
        pltpu.VMEM((SUPER, HW), jnp.int32),
        pltpu.VMEM((SUPER, HW), jnp.int32),
        pltpu.VMEM((SUPER, HW), jnp.int32),
        pltpu.VMEM((SUPER, HW), jnp.int32),
        pltpu.SemaphoreType.DMA,
        pltpu.SemaphoreType.DMA,
    ],
    compiler_params=pltpu.CompilerParams(
        needs_layout_passes=False, use_tc_tiling_on_sc=False),
)(_sc_body)


def _finalize_body(part_ref, l1_ref, out_ref):
    s2 = jnp.sum(part_ref[...])
    out_ref[0] = l1_ref[0, 0] * (1.0 / (N * D)) + s2 * (1.0 / E)


def _finalize(partials, l1):
    return pl.pallas_call(
        _finalize_body,
        out_shape=jax.ShapeDtypeStruct((1,), jnp.float32),
        in_specs=[
            pl.BlockSpec(memory_space=pltpu.VMEM),
            pl.BlockSpec(memory_space=pltpu.SMEM),
        ],
        out_specs=pl.BlockSpec(memory_space=pltpu.SMEM),
    )(partials, l1)


def kernel(x, edge_index, edge_weight, W_e, b_e, W_d, b_d):
    y_packed, l1 = _encode(x, W_e, b_e, W_d, b_d)
    row = edge_index[0]
    col = edge_index[1]
    partials = _sc_edge_loss(y_packed, row, col, edge_weight)
    out = _finalize(partials, l1)
    return out[0]
